# baseline (speedup 1.0000x reference)
import jax
import jax.numpy as jnp
from jax.experimental import pallas as pl
from jax.experimental.pallas import tpu as pltpu


def _dma_copy(t_ref, o_ref, sem):
    copy = pltpu.make_async_copy(t_ref, o_ref, sem)
    copy.start()
    copy.wait()


def kernel(x, table):
    seq = x.shape[1]
    emb = table.shape[1]
    out = pl.pallas_call(
        _dma_copy,
        in_specs=[pl.BlockSpec(memory_space=pl.ANY)],
        out_specs=pl.BlockSpec(memory_space=pl.ANY),
        out_shape=jax.ShapeDtypeStruct((seq, emb), table.dtype),
        scratch_shapes=[pltpu.SemaphoreType.DMA],
    )(table)
    return out[None, :, :]
